# baseline (speedup 1.0000x reference)
import functools

import jax
import jax.numpy as jnp
from jax import lax
from jax.experimental import pallas as pl
from jax.experimental.pallas import tpu as pltpu
from jax.experimental.pallas import tpu_sc as plsc

N_LABELS = 1000
N_DEPTHS = 3
BATCH = 16384
NC = 2
NS = 16
NW = NC * NS
L = 16
CT = (N_LABELS + 7) // 8
BT = BATCH // 128
CT_PER_W = (CT + NW - 1) // NW
BTCH = 32
NBTC = BT // BTCH


def _make_body(nd):
    def body(*args):
        tabs = args[:nd]
        leaf_hbm = args[nd]
        outs = args[nd + 1 : 2 * nd + 1]
        leaf_v, slab_v, ob0, ob1, sem0, sem1 = args[2 * nd + 1 :]
        wid = lax.axis_index("s") * NC + lax.axis_index("c")
        pltpu.sync_copy(leaf_hbm, leaf_v)
        srv = [jnp.full((L,), s * N_LABELS, jnp.int32) for s in range(8)]
        obufs = (ob0, ob1)
        sems = (sem0, sem1)

        def fill(btc, obuf):
            @plsc.parallel_loop(0, BTCH, unroll=2)
            def _(btl):
                for j in range(128 // L):
                    leaf16 = leaf_v[pl.ds((btc * BTCH + btl) * 128 + j * L, L)]
                    vs = [
                        plsc.load_gather(slab_v, [srv[s] + leaf16])
                        for s in range(8)
                    ]
                    for s in range(8):
                        obuf[btl, s, pl.ds(j * L, L)] = vs[s]

        def pair(p, _):
            for kb in range(2):
                g = p * 2 + kb
                d = g // (CT_PER_W * NBTC)
                k = (g // NBTC) % CT_PER_W
                btc = g % NBTC
                ct = wid + k * NW

                @pl.when(ct < CT)
                def _():
                    @pl.when(g >= 2)
                    def _():
                        pltpu.make_async_copy(
                            obufs[kb], outs[0].at[0, pl.ds(0, BTCH)], sems[kb]
                        ).wait()

                    for dd in range(nd):
                        @pl.when(d == dd)
                        def _(dd=dd):
                            @pl.when(btc == 0)
                            def _():
                                pltpu.sync_copy(
                                    tabs[dd].at[
                                        pl.ds(ct * (8 * N_LABELS), 8 * N_LABELS)
                                    ],
                                    slab_v,
                                )

                    fill(btc, obufs[kb])
                    for dd in range(nd):
                        @pl.when(d == dd)
                        def _(dd=dd):
                            pltpu.async_copy(
                                obufs[kb],
                                outs[dd].at[ct, pl.ds(btc * BTCH, BTCH)],
                                sems[kb],
                            )
            return 0

        lax.fori_loop(0, nd * CT_PER_W * NBTC // 2, pair, 0)
        for kb in range(2):
            pltpu.make_async_copy(
                obufs[kb], outs[0].at[0, pl.ds(0, BTCH)], sems[kb]
            ).wait()

    return body


def _make_run(nd):
    mesh = plsc.VectorSubcoreMesh(
        core_axis_name="c", subcore_axis_name="s", num_cores=NC, num_subcores=NS
    )
    out_sds = jax.ShapeDtypeStruct((CT, BT, 8, 128), jnp.float32)
    return pl.kernel(
        _make_body(nd),
        out_type=(out_sds,) * nd,
        mesh=mesh,
        scratch_types=[
            pltpu.VMEM((BATCH,), jnp.int32),
            pltpu.VMEM((8 * N_LABELS,), jnp.float32),
            pltpu.VMEM((BTCH, 8, 128), jnp.float32),
            pltpu.VMEM((BTCH, 8, 128), jnp.float32),
            pltpu.SemaphoreType.DMA,
            pltpu.SemaphoreType.DMA,
        ],
        compiler_params=pltpu.CompilerParams(
            use_tc_tiling_on_sc=False, needs_layout_passes=False
        ),
    )


@jax.jit
def kernel(labels, adversaries):
    leaf = labels[:, -1].astype(jnp.int32)
    tabs = tuple(
        jnp.transpose(adversaries[d]).reshape(N_LABELS * N_LABELS)
        for d in range(N_DEPTHS)
    )

    (oa,) = _make_run(1)(tabs[0], leaf)
    ob, oc = _make_run(2)(tabs[1], tabs[2], leaf)
    return tuple(
        o.transpose(1, 3, 0, 2).reshape(BATCH, N_LABELS) for o in (oa, ob, oc)
    )


# speedup vs baseline: 1.0004x; 1.0004x over previous
import functools

import jax
import jax.numpy as jnp
from jax import lax
from jax.experimental import pallas as pl
from jax.experimental.pallas import tpu as pltpu
from jax.experimental.pallas import tpu_sc as plsc

N_LABELS = 1000
N_DEPTHS = 3
BATCH = 16384
NC = 2
NS = 16
NW = NC * NS
L = 16
CT = (N_LABELS + 7) // 8
BT = BATCH // 128
CT_PER_W = (CT + NW - 1) // NW
BTCH = 32
NBTC = BT // BTCH


def _make_body(nd):
    def body(*args):
        tabs = args[:nd]
        leaf_hbm = args[nd]
        outs = args[nd + 1 : 2 * nd + 1]
        leaf_v, slab_v, ob0, ob1, sem0, sem1 = args[2 * nd + 1 :]
        wid = lax.axis_index("s") * NC + lax.axis_index("c")
        pltpu.sync_copy(leaf_hbm, leaf_v)
        srv = [jnp.full((L,), s * N_LABELS, jnp.int32) for s in range(8)]
        obufs = (ob0, ob1)
        sems = (sem0, sem1)

        def fill(btc, obuf):
            @plsc.parallel_loop(0, BTCH, unroll=2)
            def _(btl):
                for j in range(128 // L):
                    leaf16 = leaf_v[pl.ds((btc * BTCH + btl) * 128 + j * L, L)]
                    vs = [
                        plsc.load_gather(slab_v, [srv[s] + leaf16])
                        for s in range(8)
                    ]
                    for s in range(8):
                        obuf[btl, s, pl.ds(j * L, L)] = vs[s]

        def pair(p, _):
            for kb in range(2):
                g = p * 2 + kb
                d = g // (CT_PER_W * NBTC)
                k = (g // NBTC) % CT_PER_W
                btc = g % NBTC
                ct = wid + k * NW

                @pl.when(ct < CT)
                def _():
                    @pl.when(g >= 2)
                    def _():
                        pltpu.make_async_copy(
                            obufs[kb], outs[0].at[0, pl.ds(0, BTCH)], sems[kb]
                        ).wait()

                    for dd in range(nd):
                        @pl.when(d == dd)
                        def _(dd=dd):
                            @pl.when(btc == 0)
                            def _():
                                pltpu.sync_copy(
                                    tabs[dd].at[
                                        pl.ds(ct * (8 * N_LABELS), 8 * N_LABELS)
                                    ],
                                    slab_v,
                                )

                    fill(btc, obufs[kb])
                    for dd in range(nd):
                        @pl.when(d == dd)
                        def _(dd=dd):
                            pltpu.async_copy(
                                obufs[kb],
                                outs[dd].at[ct, pl.ds(btc * BTCH, BTCH)],
                                sems[kb],
                            )
            return 0

        lax.fori_loop(0, nd * CT_PER_W * NBTC // 2, pair, 0)
        for kb in range(2):
            pltpu.make_async_copy(
                obufs[kb], outs[0].at[0, pl.ds(0, BTCH)], sems[kb]
            ).wait()

    return body


def _make_run(nd):
    mesh = plsc.VectorSubcoreMesh(
        core_axis_name="c", subcore_axis_name="s", num_cores=NC, num_subcores=NS
    )
    out_sds = jax.ShapeDtypeStruct((CT, BT, 8, 128), jnp.float32)
    return pl.kernel(
        _make_body(nd),
        out_type=(out_sds,) * nd,
        mesh=mesh,
        scratch_types=[
            pltpu.VMEM((BATCH,), jnp.int32),
            pltpu.VMEM((8 * N_LABELS,), jnp.float32),
            pltpu.VMEM((BTCH, 8, 128), jnp.float32),
            pltpu.VMEM((BTCH, 8, 128), jnp.float32),
            pltpu.SemaphoreType.DMA,
            pltpu.SemaphoreType.DMA,
        ],
        compiler_params=pltpu.CompilerParams(
            use_tc_tiling_on_sc=False, needs_layout_passes=False
        ),
    )


@jax.jit
def kernel(labels, adversaries):
    leaf = labels[:, -1].astype(jnp.int32)
    def tp(d):
        return jnp.transpose(adversaries[d]).reshape(N_LABELS * N_LABELS)

    (oa,) = _make_run(1)(tp(0), leaf)
    ob, oc = _make_run(2)(tp(1), tp(2), leaf)
    return tuple(
        o.transpose(1, 3, 0, 2).reshape(BATCH, N_LABELS) for o in (oa, ob, oc)
    )


# speedup vs baseline: 1.0240x; 1.0235x over previous
import functools

import jax
import jax.numpy as jnp
from jax import lax
from jax.experimental import pallas as pl
from jax.experimental.pallas import tpu as pltpu
from jax.experimental.pallas import tpu_sc as plsc

N_LABELS = 1000
N_DEPTHS = 3
BATCH = 16384
NC = 2
NS = 16
NW = NC * NS
L = 16
CT = (N_LABELS + 7) // 8
BT = BATCH // 128
CT_PER_W = (CT + NW - 1) // NW
BTCH = 32
NBTC = BT // BTCH


def _make_body(nd):
    def body(*args):
        tabs = args[:nd]
        leaf_hbm = args[nd]
        outs = args[nd + 1 : 2 * nd + 1]
        leaf_v, slab_v, ob0, ob1, sem0, sem1 = args[2 * nd + 1 :]
        wid = lax.axis_index("s") * NC + lax.axis_index("c")
        pltpu.sync_copy(leaf_hbm, leaf_v)
        srv = [jnp.full((L,), s * N_LABELS, jnp.int32) for s in range(8)]
        obufs = (ob0, ob1)
        sems = (sem0, sem1)

        def fill(btc, obuf):
            @plsc.parallel_loop(0, BTCH, unroll=2)
            def _(btl):
                for j in range(128 // L):
                    leaf16 = leaf_v[pl.ds((btc * BTCH + btl) * 128 + j * L, L)]
                    vs = [
                        plsc.load_gather(slab_v, [srv[s] + leaf16])
                        for s in range(8)
                    ]
                    for s in range(8):
                        obuf[btl, s, pl.ds(j * L, L)] = vs[s]

        def pair(p, _):
            for kb in range(2):
                g = p * 2 + kb
                d = g // (CT_PER_W * NBTC)
                k = (g // NBTC) % CT_PER_W
                btc = g % NBTC
                ct = wid + k * NW

                @pl.when(ct < CT)
                def _():
                    @pl.when(g >= 2)
                    def _():
                        pltpu.make_async_copy(
                            obufs[kb], outs[0].at[0, pl.ds(0, BTCH)], sems[kb]
                        ).wait()

                    for dd in range(nd):
                        @pl.when(d == dd)
                        def _(dd=dd):
                            @pl.when(btc == 0)
                            def _():
                                pltpu.sync_copy(
                                    tabs[dd].at[
                                        pl.ds(ct * (8 * N_LABELS), 8 * N_LABELS)
                                    ],
                                    slab_v,
                                )

                    fill(btc, obufs[kb])
                    for dd in range(nd):
                        @pl.when(d == dd)
                        def _(dd=dd):
                            pltpu.async_copy(
                                obufs[kb],
                                outs[dd].at[ct, pl.ds(btc * BTCH, BTCH)],
                                sems[kb],
                            )
            return 0

        lax.fori_loop(0, nd * CT_PER_W * NBTC // 2, pair, 0)
        for kb in range(2):
            pltpu.make_async_copy(
                obufs[kb], outs[0].at[0, pl.ds(0, BTCH)], sems[kb]
            ).wait()

    return body


def _make_run(nd):
    mesh = plsc.VectorSubcoreMesh(
        core_axis_name="c", subcore_axis_name="s", num_cores=NC, num_subcores=NS
    )
    out_sds = jax.ShapeDtypeStruct((CT, BT, 8, 128), jnp.float32)
    return pl.kernel(
        _make_body(nd),
        out_type=(out_sds,) * nd,
        mesh=mesh,
        scratch_types=[
            pltpu.VMEM((BATCH,), jnp.int32),
            pltpu.VMEM((8 * N_LABELS,), jnp.float32),
            pltpu.VMEM((BTCH, 8, 128), jnp.float32),
            pltpu.VMEM((BTCH, 8, 128), jnp.float32),
            pltpu.SemaphoreType.DMA,
            pltpu.SemaphoreType.DMA,
        ],
        compiler_params=pltpu.CompilerParams(
            use_tc_tiling_on_sc=False, needs_layout_passes=False
        ),
    )


@jax.jit
def kernel(labels, adversaries):
    leaf = labels[:, -1].astype(jnp.int32)
    def tp(d):
        return jnp.transpose(adversaries[d]).reshape(N_LABELS * N_LABELS)

    oa, ob, oc = _make_run(N_DEPTHS)(tp(0), tp(1), tp(2), leaf)
    return tuple(
        o.transpose(1, 3, 0, 2).reshape(BATCH, N_LABELS) for o in (oa, ob, oc)
    )
